# baseline (speedup 1.0000x reference)
import functools

import jax
import jax.numpy as jnp
from jax import lax
from jax.experimental import pallas as pl
from jax.experimental.pallas import tpu as pltpu
from jax.experimental.pallas import tpu_sc as plsc

GAMMA = 0.5
LANES = 16
GATHER_CHUNK = 128


def _permute(v, idx16):
    dnums = lax.GatherDimensionNumbers(
        offset_dims=(), collapsed_slice_dims=(0,), start_index_map=(0,))
    return lax.gather(v, idx16[:, None], dnums, slice_sizes=(1,),
                      mode=lax.GatherScatterMode.PROMISE_IN_BOUNDS)


@functools.partial(jax.jit, static_argnums=(2, 3))
def _sc_rbf(x, idx, batch, dim):
    info = plsc.get_sparse_core_info()
    num_cores, num_subcores = info.num_cores, info.num_subcores
    num_workers = num_cores * num_subcores
    bpw = batch // num_workers

    mesh = plsc.VectorSubcoreMesh(core_axis_name="c", subcore_axis_name="s")

    @functools.partial(
        pl.kernel,
        mesh=mesh,
        out_type=jax.ShapeDtypeStruct((batch,), jnp.float32),
        compiler_params=pltpu.CompilerParams(use_tc_tiling_on_sc=False),
        scratch_types=[
            pltpu.VMEM((bpw,), jnp.int32),
            pltpu.VMEM((bpw, dim), jnp.float32),
            pltpu.VMEM((bpw, dim), jnp.float32),
            pltpu.VMEM((bpw,), jnp.float32),
        ]
        + [pltpu.SemaphoreType.DMA] * (2 * (bpw // GATHER_CHUNK)),
    )
    def k(x_hbm, idx_hbm, out_hbm, idx_v, x_v, cent_v, sums_v, *sems):
        nchunks = bpw // GATHER_CHUNK
        gsems, xsems = sems[:nchunks], sems[nchunks:]
        wid = lax.axis_index("s") * num_cores + lax.axis_index("c")
        base = wid * bpw

        pltpu.sync_copy(idx_hbm.at[pl.ds(base, bpw)], idx_v)

        copies = []
        for c in range(nchunks):
            lo = c * GATHER_CHUNK
            copies.append((
                pltpu.async_copy(
                    x_hbm.at[idx_v.at[pl.ds(lo, GATHER_CHUNK)]],
                    cent_v.at[pl.ds(lo, GATHER_CHUNK), :],
                    gsems[c],
                ),
                pltpu.async_copy(
                    x_hbm.at[pl.ds(base + lo, GATHER_CHUNK), :],
                    x_v.at[pl.ds(lo, GATHER_CHUNK), :],
                    xsems[c],
                ),
            ))

        lane = lax.iota(jnp.int32, LANES)
        perms = [lane ^ (1 << k) for k in range(4)]

        def grp(g, carry):
            off = pl.multiple_of(g * LANES, LANES)
            out_vec = jnp.zeros((LANES,), jnp.float32)
            for l in range(LANES):
                r = off + l
                acc = jnp.zeros((LANES,), jnp.float32)
                for c4 in range(dim // LANES):
                    dx = (x_v[r, pl.ds(c4 * LANES, LANES)]
                          - cent_v[r, pl.ds(c4 * LANES, LANES)])
                    acc = acc + dx * dx
                for p in perms:
                    acc = acc + _permute(acc, p)
                out_vec = jnp.where(lane == l, acc, out_vec)
            sums_v[pl.ds(off, LANES)] = jnp.exp(-GAMMA * out_vec)
            return carry

        gpc = GATHER_CHUNK // LANES
        for c in range(nchunks):
            copies[c][0].wait()
            copies[c][1].wait()
            lax.fori_loop(c * gpc, (c + 1) * gpc, grp, 0)

        pltpu.sync_copy(sums_v, out_hbm.at[pl.ds(base, bpw)])

    return k(x, idx)


def kernel(x, rng):
    if x.ndim == 1:
        x = x.reshape(-1, 1)
    batch, dim = x.shape
    centers_idx = jax.random.randint(rng, (batch,), 0, batch)
    return _sc_rbf(x, centers_idx.astype(jnp.int32), batch, dim)


# speedup vs baseline: 1.1201x; 1.1201x over previous
import functools

import jax
import jax.numpy as jnp
from jax import lax
from jax.experimental import pallas as pl
from jax.experimental.pallas import tpu as pltpu
from jax.experimental.pallas import tpu_sc as plsc

GAMMA = 0.5
LANES = 16
GATHER_CHUNK = 128

_ROT = (13, 15, 26, 6, 17, 29, 16, 24)


def _permute(v, idx16):
    dnums = lax.GatherDimensionNumbers(
        offset_dims=(), collapsed_slice_dims=(0,), start_index_map=(0,))
    return lax.gather(v, idx16[:, None], dnums, slice_sizes=(1,),
                      mode=lax.GatherScatterMode.PROMISE_IN_BOUNDS)


def _rotl(x, r):
    return (x << jnp.uint32(r)) | (x >> jnp.uint32(32 - r))


def _threefry_pair(k0, k1, x0, x1):
    ks2 = k0 ^ k1 ^ jnp.uint32(0x1BD11BDA)
    ks = (k0, k1, ks2)
    x0 = x0 + ks[0]
    x1 = x1 + ks[1]
    for g in range(5):
        rots = _ROT[0:4] if g % 2 == 0 else _ROT[4:8]
        for r in rots:
            x0 = x0 + x1
            x1 = _rotl(x1, r)
            x1 = x0 ^ x1
        x0 = x0 + ks[(g + 1) % 3]
        x1 = x1 + ks[(g + 2) % 3] + jnp.uint32(g + 1)
    return x0, x1


@functools.partial(jax.jit, static_argnums=(2, 3))
def _sc_rbf(x, key_words, batch, dim):
    info = plsc.get_sparse_core_info()
    num_cores, num_subcores = info.num_cores, info.num_subcores
    num_workers = num_cores * num_subcores
    bpw = batch // num_workers
    nchunks = bpw // GATHER_CHUNK
    mask = jnp.uint32(batch - 1)

    mesh = plsc.VectorSubcoreMesh(core_axis_name="c", subcore_axis_name="s")

    @functools.partial(
        pl.kernel,
        mesh=mesh,
        out_type=jax.ShapeDtypeStruct((batch,), jnp.float32),
        compiler_params=pltpu.CompilerParams(use_tc_tiling_on_sc=False),
        scratch_types=[
            pltpu.VMEM((LANES,), jnp.uint32),
            pltpu.VMEM((bpw,), jnp.int32),
            pltpu.VMEM((bpw, dim), jnp.float32),
            pltpu.VMEM((bpw, dim), jnp.float32),
            pltpu.VMEM((bpw,), jnp.float32),
        ]
        + [pltpu.SemaphoreType.DMA] * (nchunks + 1),
    )
    def k(x_hbm, key_hbm, out_hbm, key_v, idx_v, x_v, cent_v, sums_v, *sems):
        gsems, xsem = sems[:nchunks], sems[nchunks]
        wid = lax.axis_index("s") * num_cores + lax.axis_index("c")
        base = wid * bpw

        xcopy = pltpu.async_copy(
            x_hbm.at[pl.ds(base, bpw), :], x_v, xsem)

        pltpu.sync_copy(key_hbm, key_v)
        lane = lax.iota(jnp.int32, LANES)
        zeros_i = jnp.zeros((LANES,), jnp.int32)
        kv = key_v[...]
        k0 = _permute(kv, zeros_i)
        k1 = _permute(kv, zeros_i + 1)
        b0, b1 = _threefry_pair(
            k0, k1, jnp.zeros((LANES,), jnp.uint32), lane.astype(jnp.uint32))
        k2a = _permute(b0, zeros_i + 1)
        k2b = _permute(b1, zeros_i + 1)

        zeros_u = jnp.zeros((LANES,), jnp.uint32)
        lane_u = lane.astype(jnp.uint32)

        copies = []
        for c in range(nchunks):
            lo = c * GATHER_CHUNK

            def tf(j, carry):
                off = lo + j * LANES
                ctr = jnp.uint32(base) + jnp.uint32(off) + lane_u
                o0, o1 = _threefry_pair(k2a, k2b, zeros_u, ctr)
                bits = o0 ^ o1
                idx_v[pl.ds(off, LANES)] = (bits & mask).astype(jnp.int32)
                return carry

            lax.fori_loop(0, GATHER_CHUNK // LANES, tf, 0)
            copies.append(
                pltpu.async_copy(
                    x_hbm.at[idx_v.at[pl.ds(lo, GATHER_CHUNK)]],
                    cent_v.at[pl.ds(lo, GATHER_CHUNK), :],
                    gsems[c],
                ))

        perms = [lane ^ (1 << k) for k in range(4)]

        def grp(g, carry):
            off = pl.multiple_of(g * LANES, LANES)
            out_vec = jnp.zeros((LANES,), jnp.float32)
            for l in range(LANES):
                r = off + l
                acc = jnp.zeros((LANES,), jnp.float32)
                for c4 in range(dim // LANES):
                    dx = (x_v[r, pl.ds(c4 * LANES, LANES)]
                          - cent_v[r, pl.ds(c4 * LANES, LANES)])
                    acc = acc + dx * dx
                for p in perms:
                    acc = acc + _permute(acc, p)
                out_vec = jnp.where(lane == l, acc, out_vec)
            sums_v[pl.ds(off, LANES)] = jnp.exp(-GAMMA * out_vec)
            return carry

        xcopy.wait()
        gpc = GATHER_CHUNK // LANES
        for c in range(nchunks):
            copies[c].wait()
            lax.fori_loop(c * gpc, (c + 1) * gpc, grp, 0)

        pltpu.sync_copy(sums_v, out_hbm.at[pl.ds(base, bpw)])

    return k(x, key_words)


def kernel(x, rng):
    if x.ndim == 1:
        x = x.reshape(-1, 1)
    batch, dim = x.shape
    key_words = jnp.zeros((LANES,), jnp.uint32).at[:2].set(
        jax.random.key_data(rng))
    return _sc_rbf(x, key_words, batch, dim)
